# baseline (speedup 1.0000x reference)
import jax
import jax.numpy as jnp
import numpy as np
from jax.experimental import pallas as pl
from jax.experimental.pallas import tpu as pltpu

_B = 8
_NP = 5000
_PAD = 5120
_R = 40
_C = 128
_NGT = 100
_P = 66
_NEG = 134
_ROIS = 200
_MH = 256
_MS = 28
_STD = (0.1, 0.1, 0.2, 0.2)
_NEGINF = -1e9


def _s11(x):
    return jnp.reshape(x, (1, 1))


def _stage1_kernel(props_ref, gt_ref, cls_ref, rois_ref, clsout_ref,
                   deltas_ref, asgn_ref):
    p0 = props_ref[0, 0]
    p1 = props_ref[0, 1]
    p2 = props_ref[0, 2]
    p3 = props_ref[0, 3]
    area_p = (p2 - p0) * (p3 - p1)
    vp = ((jnp.abs(p0) > 0.0) | (jnp.abs(p1) > 0.0)
          | (jnp.abs(p2) > 0.0) | (jnp.abs(p3) > 0.0))

    def gt_body(g, carry):
        m, asgn = carry
        g0 = gt_ref[0, pl.ds(g, 1), 0:1]
        g1 = gt_ref[0, pl.ds(g, 1), 1:2]
        g2 = gt_ref[0, pl.ds(g, 1), 2:3]
        g3 = gt_ref[0, pl.ds(g, 1), 3:4]
        cv = cls_ref[0, pl.ds(g, 1), 0:1] > 0
        yy1 = jnp.maximum(p0, g0)
        xx1 = jnp.maximum(p1, g1)
        yy2 = jnp.minimum(p2, g2)
        xx2 = jnp.minimum(p3, g3)
        inter = jnp.maximum(yy2 - yy1, 0.0) * jnp.maximum(xx2 - xx1, 0.0)
        area_g = (g2 - g0) * (g3 - g1)
        union = jnp.maximum(area_p + area_g - inter, 1e-8)
        iou = jnp.where(cv, inter / union, -1.0)
        upd = iou > m
        m = jnp.where(upd, iou, m)
        asgn = jnp.where(upd, g, asgn)
        return m, asgn

    m0 = jnp.full((_R, _C), _NEGINF, jnp.float32)
    a0 = jnp.zeros((_R, _C), jnp.int32)
    m, asgn = jax.lax.fori_loop(0, _NGT, gt_body, (m0, a0))

    fiota = (jax.lax.broadcasted_iota(jnp.int32, (_R, _C), 0) * _C
             + jax.lax.broadcasted_iota(jnp.int32, (_R, _C), 1))

    kp = jnp.where((m >= 0.5) & vp, m, _NEGINF)
    kn = jnp.where((m < 0.5) & vp, 1.0 - m, _NEGINF)

    rois_ref[...] = jnp.zeros_like(rois_ref)
    clsout_ref[...] = jnp.zeros_like(clsout_ref)
    deltas_ref[...] = jnp.zeros_like(deltas_ref)

    def pos_body(i, kp):
        mx = jnp.max(kp)
        idx = jnp.min(jnp.where(kp == mx, fiota, jnp.int32(2**30)))
        sel = fiota == idx
        valid = mx >= 0.5
        vf = jnp.where(valid, 1.0, 0.0)
        a = jnp.sum(jnp.where(sel, asgn, 0))
        q0 = jnp.sum(jnp.where(sel, p0, 0.0))
        q1 = jnp.sum(jnp.where(sel, p1, 0.0))
        q2 = jnp.sum(jnp.where(sel, p2, 0.0))
        q3 = jnp.sum(jnp.where(sel, p3, 0.0))
        g0 = gt_ref[0, pl.ds(a, 1), 0:1][0, 0]
        g1 = gt_ref[0, pl.ds(a, 1), 1:2][0, 0]
        g2 = gt_ref[0, pl.ds(a, 1), 2:3][0, 0]
        g3 = gt_ref[0, pl.ds(a, 1), 3:4][0, 0]
        c = cls_ref[0, pl.ds(a, 1), 0:1][0, 0]
        h = q2 - q0
        w = q3 - q1
        hs = jnp.where(valid, h, 1.0)
        ws = jnp.where(valid, w, 1.0)
        ghs = jnp.where(valid, g2 - g0, 1.0)
        gws = jnp.where(valid, g3 - g1, 1.0)
        cy = q0 + 0.5 * h
        cx = q1 + 0.5 * w
        gcy = g0 + 0.5 * (g2 - g0)
        gcx = g1 + 0.5 * (g3 - g1)
        d0 = vf * ((gcy - cy) / hs) / _STD[0]
        d1 = vf * ((gcx - cx) / ws) / _STD[1]
        d2 = vf * jnp.log(ghs / hs) / _STD[2]
        d3 = vf * jnp.log(gws / ws) / _STD[3]
        rois_ref[0, pl.ds(i, 1), 0:1] = _s11(q0 * vf)
        rois_ref[0, pl.ds(i, 1), 1:2] = _s11(q1 * vf)
        rois_ref[0, pl.ds(i, 1), 2:3] = _s11(q2 * vf)
        rois_ref[0, pl.ds(i, 1), 3:4] = _s11(q3 * vf)
        deltas_ref[0, pl.ds(i, 1), 0:1] = _s11(d0)
        deltas_ref[0, pl.ds(i, 1), 1:2] = _s11(d1)
        deltas_ref[0, pl.ds(i, 1), 2:3] = _s11(d2)
        deltas_ref[0, pl.ds(i, 1), 3:4] = _s11(d3)
        clsout_ref[0, pl.ds(i, 1), 0:1] = _s11(jnp.where(valid, c, 0))
        asgn_ref[0, pl.ds(i, 1), 0:1] = _s11(jnp.where(valid, a, -1))
        return jnp.where(sel, _NEGINF, kp)

    jax.lax.fori_loop(0, _P, pos_body, kp)

    def neg_body(i, kn):
        mx = jnp.max(kn)
        idx = jnp.min(jnp.where(kn == mx, fiota, jnp.int32(2**30)))
        sel = fiota == idx
        vf = jnp.where(mx >= 0.5, 1.0, 0.0)
        q0 = jnp.sum(jnp.where(sel, p0, 0.0))
        q1 = jnp.sum(jnp.where(sel, p1, 0.0))
        q2 = jnp.sum(jnp.where(sel, p2, 0.0))
        q3 = jnp.sum(jnp.where(sel, p3, 0.0))
        rois_ref[0, pl.ds(_P + i, 1), 0:1] = _s11(q0 * vf)
        rois_ref[0, pl.ds(_P + i, 1), 1:2] = _s11(q1 * vf)
        rois_ref[0, pl.ds(_P + i, 1), 2:3] = _s11(q2 * vf)
        rois_ref[0, pl.ds(_P + i, 1), 3:4] = _s11(q3 * vf)
        return jnp.where(sel, _NEGINF, kn)

    jax.lax.fori_loop(0, _NEG, neg_body, kn)


def _stage2_kernel(asgn_ref, mask_ref, rois_ref, out_ref):
    b = pl.program_id(0)
    i = pl.program_id(1)
    valid = asgn_ref[b, i] >= 0
    b0 = rois_ref[0, pl.ds(i, 1), 0:1]
    b1 = rois_ref[0, pl.ds(i, 1), 1:2]
    b2 = rois_ref[0, pl.ds(i, 1), 2:3]
    b3 = rois_ref[0, pl.ds(i, 1), 3:4]
    oy = jax.lax.broadcasted_iota(jnp.int32, (_MS, 1), 0).astype(
        jnp.float32) / (_MS - 1.0)
    py = (b0 + (b2 - b0) * oy) * (_MH - 1.0)
    y0f = jnp.clip(jnp.floor(py), 0.0, _MH - 1.0)
    wy = jnp.clip(py - y0f, 0.0, 1.0)
    y1f = jnp.minimum(y0f + 1.0, _MH - 1.0)
    ioy = jax.lax.broadcasted_iota(jnp.int32, (_MS, _MH), 1).astype(
        jnp.float32)
    wym = (jnp.where(ioy == y0f, 1.0 - wy, 0.0)
           + jnp.where(ioy == y1f, wy, 0.0))
    mimg = mask_ref[0, 0].astype(jnp.float32)
    t = jnp.dot(wym, mimg, preferred_element_type=jnp.float32,
                precision=jax.lax.Precision.HIGHEST)
    ox = jax.lax.broadcasted_iota(jnp.int32, (1, _MS), 1).astype(
        jnp.float32) / (_MS - 1.0)
    px = (b1 + (b3 - b1) * ox) * (_MH - 1.0)
    x0f = jnp.clip(jnp.floor(px), 0.0, _MH - 1.0)
    wx = jnp.clip(px - x0f, 0.0, 1.0)
    x1f = jnp.minimum(x0f + 1.0, _MH - 1.0)
    iox = jax.lax.broadcasted_iota(jnp.int32, (_MH, _MS), 0).astype(
        jnp.float32)
    wxm = (jnp.where(iox == x0f, 1.0 - wx, 0.0)
           + jnp.where(iox == x1f, wx, 0.0))
    crop = jnp.dot(t, wxm, preferred_element_type=jnp.float32,
                   precision=jax.lax.Precision.HIGHEST)
    out_ref[0, 0] = jnp.round(crop) * jnp.where(valid, 1.0, 0.0)


def kernel(proposals, prior_class_ids, prior_boxes, prior_masks):
    propsp = jnp.pad(proposals, ((0, 0), (0, _PAD - _NP), (0, 0)))
    props_c = propsp.transpose(0, 2, 1).reshape(_B, 4, _R, _C)
    cls3 = prior_class_ids.astype(jnp.int32).reshape(_B, _NGT, 1)

    rois, cls_o, deltas, asgn = pl.pallas_call(
        _stage1_kernel,
        grid=(_B,),
        in_specs=[
            pl.BlockSpec((1, 4, _R, _C), lambda b: (b, 0, 0, 0)),
            pl.BlockSpec((1, _NGT, 4), lambda b: (b, 0, 0)),
            pl.BlockSpec((1, _NGT, 1), lambda b: (b, 0, 0)),
        ],
        out_specs=[
            pl.BlockSpec((1, _ROIS, 4), lambda b: (b, 0, 0)),
            pl.BlockSpec((1, _ROIS, 1), lambda b: (b, 0, 0)),
            pl.BlockSpec((1, _ROIS, 4), lambda b: (b, 0, 0)),
            pl.BlockSpec((1, _P, 1), lambda b: (b, 0, 0)),
        ],
        out_shape=[
            jax.ShapeDtypeStruct((_B, _ROIS, 4), jnp.float32),
            jax.ShapeDtypeStruct((_B, _ROIS, 1), jnp.int32),
            jax.ShapeDtypeStruct((_B, _ROIS, 4), jnp.float32),
            jax.ShapeDtypeStruct((_B, _P, 1), jnp.int32),
        ],
    )(props_c, prior_boxes, cls3)

    masks_t = prior_masks.transpose(0, 3, 1, 2)
    asgn2 = asgn.reshape(_B, _P)

    crops = pl.pallas_call(
        _stage2_kernel,
        grid_spec=pltpu.PrefetchScalarGridSpec(
            num_scalar_prefetch=1,
            grid=(_B, _P),
            in_specs=[
                pl.BlockSpec(
                    (1, 1, _MH, _MH),
                    lambda b, i, a: (b, jnp.maximum(a[b, i], 0), 0, 0)),
                pl.BlockSpec((1, _ROIS, 4), lambda b, i, a: (b, 0, 0)),
            ],
            out_specs=pl.BlockSpec(
                (1, 1, _MS, _MS), lambda b, i, a: (b, i, 0, 0)),
        ),
        out_shape=jax.ShapeDtypeStruct((_B, _P, _MS, _MS), jnp.float32),
    )(asgn2, masks_t, rois)

    masks = jnp.concatenate(
        [crops, jnp.zeros((_B, _NEG, _MS, _MS), jnp.float32)], axis=1)
    return rois, cls_o.reshape(_B, _ROIS), deltas, masks


# speedup vs baseline: 1.1096x; 1.1096x over previous
import jax
import jax.numpy as jnp
import numpy as np
from jax.experimental import pallas as pl
from jax.experimental.pallas import tpu as pltpu

_B = 8
_NP = 5000
_PAD = 5120
_R = 40
_C = 128
_NGT = 100
_P = 66
_NEG = 134
_ROIS = 200
_MH = 256
_MS = 28
_STD = (0.1, 0.1, 0.2, 0.2)
_NEGINF = -1e9
_TPB = 11


def _s11(x):
    return jnp.reshape(x, (1, 1))


def _stage1_kernel(props_ref, gt_ref, cls_ref, props2_ref, rois_ref,
                   clsout_ref, deltas_ref, asgn_ref, sidx_ref, sa_ref):
    p0 = props_ref[0, 0]
    p1 = props_ref[0, 1]
    p2 = props_ref[0, 2]
    p3 = props_ref[0, 3]
    area_p = (p2 - p0) * (p3 - p1)
    vp = ((jnp.abs(p0) > 0.0) | (jnp.abs(p1) > 0.0)
          | (jnp.abs(p2) > 0.0) | (jnp.abs(p3) > 0.0))

    def gt_body(g, carry):
        m, asgn = carry
        g0 = gt_ref[0, pl.ds(g, 1), 0:1]
        g1 = gt_ref[0, pl.ds(g, 1), 1:2]
        g2 = gt_ref[0, pl.ds(g, 1), 2:3]
        g3 = gt_ref[0, pl.ds(g, 1), 3:4]
        cv = cls_ref[0, pl.ds(g, 1), 0:1] > 0
        yy1 = jnp.maximum(p0, g0)
        xx1 = jnp.maximum(p1, g1)
        yy2 = jnp.minimum(p2, g2)
        xx2 = jnp.minimum(p3, g3)
        inter = jnp.maximum(yy2 - yy1, 0.0) * jnp.maximum(xx2 - xx1, 0.0)
        area_g = (g2 - g0) * (g3 - g1)
        union = jnp.maximum(area_p + area_g - inter, 1e-8)
        iou = jnp.where(cv, inter / union, -1.0)
        upd = iou > m
        m = jnp.where(upd, iou, m)
        asgn = jnp.where(upd, g, asgn)
        return m, asgn

    m0 = jnp.full((_R, _C), _NEGINF, jnp.float32)
    a0 = jnp.zeros((_R, _C), jnp.int32)
    m, asgn = jax.lax.fori_loop(0, _NGT, gt_body, (m0, a0))

    fiota = (jax.lax.broadcasted_iota(jnp.int32, (_R, _C), 0) * _C
             + jax.lax.broadcasted_iota(jnp.int32, (_R, _C), 1))

    kp = jnp.where((m >= 0.5) & vp, m, _NEGINF)
    kn = jnp.where((m < 0.5) & vp, 1.0 - m, _NEGINF)

    def pos_sel(i, kp):
        mx = jnp.max(kp)
        idx = jnp.min(jnp.where(kp == mx, fiota, jnp.int32(2**30)))
        sel = fiota == idx
        a = jnp.sum(jnp.where(sel, asgn, 0))
        sidx_ref[i] = jnp.where(mx >= 0.5, idx, -1)
        sa_ref[i] = a
        return jnp.where(sel, _NEGINF, kp)

    jax.lax.fori_loop(0, _P, pos_sel, kp)

    def neg_sel(i, kn):
        mx = jnp.max(kn)
        idx = jnp.min(jnp.where(kn == mx, fiota, jnp.int32(2**30)))
        sel = fiota == idx
        sidx_ref[_P + i] = jnp.where(mx >= 0.5, idx, -1)
        return jnp.where(sel, _NEGINF, kn)

    jax.lax.fori_loop(0, _NEG, neg_sel, kn)

    rois_ref[...] = jnp.zeros_like(rois_ref)
    clsout_ref[...] = jnp.zeros_like(clsout_ref)
    deltas_ref[...] = jnp.zeros_like(deltas_ref)

    def pos_body(i, _):
        si = sidx_ref[i]
        valid = si >= 0
        pidx = jnp.maximum(si, 0)
        vf = jnp.where(valid, 1.0, 0.0)
        a = sa_ref[i]
        q0 = props2_ref[0, pl.ds(pidx, 1), 0:1][0, 0]
        q1 = props2_ref[0, pl.ds(pidx, 1), 1:2][0, 0]
        q2 = props2_ref[0, pl.ds(pidx, 1), 2:3][0, 0]
        q3 = props2_ref[0, pl.ds(pidx, 1), 3:4][0, 0]
        g0 = gt_ref[0, pl.ds(a, 1), 0:1][0, 0]
        g1 = gt_ref[0, pl.ds(a, 1), 1:2][0, 0]
        g2 = gt_ref[0, pl.ds(a, 1), 2:3][0, 0]
        g3 = gt_ref[0, pl.ds(a, 1), 3:4][0, 0]
        c = cls_ref[0, pl.ds(a, 1), 0:1][0, 0]
        h = q2 - q0
        w = q3 - q1
        hs = jnp.where(valid, h, 1.0)
        ws = jnp.where(valid, w, 1.0)
        ghs = jnp.where(valid, g2 - g0, 1.0)
        gws = jnp.where(valid, g3 - g1, 1.0)
        cy = q0 + 0.5 * h
        cx = q1 + 0.5 * w
        gcy = g0 + 0.5 * (g2 - g0)
        gcx = g1 + 0.5 * (g3 - g1)
        d0 = vf * ((gcy - cy) / hs) / _STD[0]
        d1 = vf * ((gcx - cx) / ws) / _STD[1]
        d2 = vf * jnp.log(ghs / hs) / _STD[2]
        d3 = vf * jnp.log(gws / ws) / _STD[3]
        rois_ref[0, pl.ds(i, 1), 0:1] = _s11(q0 * vf)
        rois_ref[0, pl.ds(i, 1), 1:2] = _s11(q1 * vf)
        rois_ref[0, pl.ds(i, 1), 2:3] = _s11(q2 * vf)
        rois_ref[0, pl.ds(i, 1), 3:4] = _s11(q3 * vf)
        deltas_ref[0, pl.ds(i, 1), 0:1] = _s11(d0)
        deltas_ref[0, pl.ds(i, 1), 1:2] = _s11(d1)
        deltas_ref[0, pl.ds(i, 1), 2:3] = _s11(d2)
        deltas_ref[0, pl.ds(i, 1), 3:4] = _s11(d3)
        clsout_ref[0, pl.ds(i, 1), 0:1] = _s11(jnp.where(valid, c, 0))
        asgn_ref[0, pl.ds(i, 1), 0:1] = _s11(jnp.where(valid, a, -1))
        return 0

    jax.lax.fori_loop(0, _P, pos_body, 0)

    def neg_body(i, _):
        si = sidx_ref[_P + i]
        valid = si >= 0
        pidx = jnp.maximum(si, 0)
        vf = jnp.where(valid, 1.0, 0.0)
        q0 = props2_ref[0, pl.ds(pidx, 1), 0:1][0, 0]
        q1 = props2_ref[0, pl.ds(pidx, 1), 1:2][0, 0]
        q2 = props2_ref[0, pl.ds(pidx, 1), 2:3][0, 0]
        q3 = props2_ref[0, pl.ds(pidx, 1), 3:4][0, 0]
        rois_ref[0, pl.ds(_P + i, 1), 0:1] = _s11(q0 * vf)
        rois_ref[0, pl.ds(_P + i, 1), 1:2] = _s11(q1 * vf)
        rois_ref[0, pl.ds(_P + i, 1), 2:3] = _s11(q2 * vf)
        rois_ref[0, pl.ds(_P + i, 1), 3:4] = _s11(q3 * vf)
        return 0

    jax.lax.fori_loop(0, _NEG, neg_body, 0)


def _crop_one(mask_ref, rois_ref, i, valid):
    b0 = rois_ref[0, pl.ds(i, 1), 0:1]
    b1 = rois_ref[0, pl.ds(i, 1), 1:2]
    b2 = rois_ref[0, pl.ds(i, 1), 2:3]
    b3 = rois_ref[0, pl.ds(i, 1), 3:4]
    oy = jax.lax.broadcasted_iota(jnp.int32, (_MS, 1), 0).astype(
        jnp.float32) / (_MS - 1.0)
    py = (b0 + (b2 - b0) * oy) * (_MH - 1.0)
    y0f = jnp.clip(jnp.floor(py), 0.0, _MH - 1.0)
    wy = jnp.clip(py - y0f, 0.0, 1.0)
    y1f = jnp.minimum(y0f + 1.0, _MH - 1.0)
    ioy = jax.lax.broadcasted_iota(jnp.int32, (_MS, _MH), 1).astype(
        jnp.float32)
    wym = (jnp.where(ioy == y0f, 1.0 - wy, 0.0)
           + jnp.where(ioy == y1f, wy, 0.0))
    mimg = mask_ref[0, 0].astype(jnp.float32)
    t = jnp.dot(wym, mimg, preferred_element_type=jnp.float32,
                precision=jax.lax.Precision.HIGHEST)
    ox = jax.lax.broadcasted_iota(jnp.int32, (1, _MS), 1).astype(
        jnp.float32) / (_MS - 1.0)
    px = (b1 + (b3 - b1) * ox) * (_MH - 1.0)
    x0f = jnp.clip(jnp.floor(px), 0.0, _MH - 1.0)
    wx = jnp.clip(px - x0f, 0.0, 1.0)
    x1f = jnp.minimum(x0f + 1.0, _MH - 1.0)
    iox = jax.lax.broadcasted_iota(jnp.int32, (_MH, _MS), 0).astype(
        jnp.float32)
    wxm = (jnp.where(iox == x0f, 1.0 - wx, 0.0)
           + jnp.where(iox == x1f, wx, 0.0))
    crop = jnp.dot(t, wxm, preferred_element_type=jnp.float32,
                   precision=jax.lax.Precision.HIGHEST)
    return jnp.round(crop) * jnp.where(valid, 1.0, 0.0)


def _stage2_kernel(asgn_ref, *refs):
    mask_refs = refs[:_TPB]
    rois_ref = refs[_TPB]
    out_ref = refs[_TPB + 1]
    b = pl.program_id(0)
    j = pl.program_id(1)
    for t in range(_TPB):
        i = j * _TPB + t
        valid = asgn_ref[b, i] >= 0
        out_ref[0, t] = _crop_one(mask_refs[t], rois_ref, i, valid)


def kernel(proposals, prior_class_ids, prior_boxes, prior_masks):
    propsp = jnp.pad(proposals, ((0, 0), (0, _PAD - _NP), (0, 0)))
    props_c = propsp.transpose(0, 2, 1).reshape(_B, 4, _R, _C)
    cls3 = prior_class_ids.astype(jnp.int32).reshape(_B, _NGT, 1)

    rois, cls_o, deltas, asgn = pl.pallas_call(
        _stage1_kernel,
        grid=(_B,),
        in_specs=[
            pl.BlockSpec((1, 4, _R, _C), lambda b: (b, 0, 0, 0)),
            pl.BlockSpec((1, _NGT, 4), lambda b: (b, 0, 0)),
            pl.BlockSpec((1, _NGT, 1), lambda b: (b, 0, 0)),
            pl.BlockSpec((1, _PAD, 4), lambda b: (b, 0, 0)),
        ],
        out_specs=[
            pl.BlockSpec((1, _ROIS, 4), lambda b: (b, 0, 0)),
            pl.BlockSpec((1, _ROIS, 1), lambda b: (b, 0, 0)),
            pl.BlockSpec((1, _ROIS, 4), lambda b: (b, 0, 0)),
            pl.BlockSpec((1, _P, 1), lambda b: (b, 0, 0)),
        ],
        out_shape=[
            jax.ShapeDtypeStruct((_B, _ROIS, 4), jnp.float32),
            jax.ShapeDtypeStruct((_B, _ROIS, 1), jnp.int32),
            jax.ShapeDtypeStruct((_B, _ROIS, 4), jnp.float32),
            jax.ShapeDtypeStruct((_B, _P, 1), jnp.int32),
        ],
        scratch_shapes=[
            pltpu.SMEM((_ROIS,), jnp.int32),
            pltpu.SMEM((_P,), jnp.int32),
        ],
    )(props_c, prior_boxes, cls3, propsp)

    masks_t = prior_masks.transpose(0, 3, 1, 2)
    asgn2 = asgn.reshape(_B, _P)

    def _mk_mask_spec(t):
        return pl.BlockSpec(
            (1, 1, _MH, _MH),
            lambda b, j, a: (b, jnp.maximum(a[b, j * _TPB + t], 0), 0, 0))

    crops = pl.pallas_call(
        _stage2_kernel,
        grid_spec=pltpu.PrefetchScalarGridSpec(
            num_scalar_prefetch=1,
            grid=(_B, _P // _TPB),
            in_specs=[_mk_mask_spec(t) for t in range(_TPB)]
            + [pl.BlockSpec((1, _ROIS, 4), lambda b, j, a: (b, 0, 0))],
            out_specs=pl.BlockSpec(
                (1, _TPB, _MS, _MS), lambda b, j, a: (b, j, 0, 0)),
        ),
        out_shape=jax.ShapeDtypeStruct((_B, _P, _MS, _MS), jnp.float32),
    )(asgn2, *([masks_t] * _TPB), rois)

    masks = jnp.concatenate(
        [crops, jnp.zeros((_B, _NEG, _MS, _MS), jnp.float32)], axis=1)
    return rois, cls_o.reshape(_B, _ROIS), deltas, masks
